# baseline (speedup 1.0000x reference)
import jax
import jax.numpy as jnp
from jax import lax
from jax.experimental import pallas as pl
from jax.experimental.pallas import tpu as pltpu
from jax.experimental.pallas import tpu_sc as plsc

N, D, B = 32768, 32, 16
FC0, FC1 = 64, 128
NC, NS = 2, 16
NW = NC * NS
CHUNK = 256
NBUF = 2
NJ = N // (CHUNK * NW)
UNROLL = 4


def _sc_segment_sums_body(x_hbm, len_hbm, part_hbm,
                          b0, b1, len_v, part_v, s0, s1):
    c = lax.axis_index("c")
    s = lax.axis_index("s")
    wid = c * NS + s
    bufs = [b0, b1]
    sems = [s0, s1]

    zf = jnp.zeros((16,), jnp.float32)
    for b in range(B):
        part_v[b, 0:16] = zf
        part_v[b, 16:32] = zf

    pltpu.sync_copy(len_hbm, len_v)
    lv = len_v[:]
    cbs = []
    run = jnp.int32(0)
    for b in range(B):
        run = run + lv[b]
        cbs.append(run)
    total = cbs[B - 1]

    cbases = [(j * NW + wid) * CHUNK for j in range(NJ)]

    def issue(j):
        @pl.when(cbases[j] < total)
        def _():
            pltpu.async_copy(x_hbm.at[pl.ds(cbases[j], CHUNK)],
                             bufs[j % NBUF], sems[j % NBUF])

    for j in range(min(NBUF, NJ)):
        issue(j)

    for j in range(NJ):
        x_v = bufs[j % NBUF]

        @pl.when(cbases[j] < total)
        def _(j=j, x_v=x_v):
            pltpu.make_async_copy(x_hbm.at[pl.ds(cbases[j], CHUNK)],
                                  x_v, sems[j % NBUF]).wait()
            hbase = cbases[j]
            hi_h = hbase + CHUNK
            for b in range(B):
                lo_g = cbs[b - 1] if b else jnp.int32(0)
                lo = jnp.clip(lo_g, hbase, hi_h) - hbase
                hi = jnp.clip(cbs[b], hbase, hi_h) - hbase
                n = hi - lo

                @pl.when(n > 0)
                def _(b=b, lo=lo, hi=hi, n=n, x_v=x_v):
                    def u_body(t, accs, lo=lo):
                        r = lo + t * UNROLL
                        out = []
                        for u in range(UNROLL):
                            a0 = accs[2 * u] + x_v[r + u, 0:16]
                            a1 = accs[2 * u + 1] + x_v[r + u, 16:32]
                            out.extend((a0, a1))
                        return tuple(out)

                    n_main = n // UNROLL
                    accs = lax.fori_loop(0, n_main, u_body, (zf,) * (2 * UNROLL))

                    def r_body(r, accs2):
                        return (accs2[0] + x_v[r, 0:16],
                                accs2[1] + x_v[r, 16:32])

                    tail = lax.fori_loop(lo + n_main * UNROLL, hi, r_body,
                                         (zf, zf))

                    acc_lo = (accs[0] + accs[2]) + (accs[4] + accs[6]) + tail[0]
                    acc_hi = (accs[1] + accs[3]) + (accs[5] + accs[7]) + tail[1]
                    part_v[b, 0:16] = part_v[b, 0:16] + acc_lo
                    part_v[b, 16:32] = part_v[b, 16:32] + acc_hi

        if j + NBUF < NJ:
            issue(j + NBUF)

    pltpu.sync_copy(part_v, part_hbm.at[wid])


_sc_segment_sums = pl.kernel(
    _sc_segment_sums_body,
    out_type=jax.ShapeDtypeStruct((NW, B, D), jnp.float32),
    mesh=plsc.VectorSubcoreMesh(core_axis_name="c", subcore_axis_name="s"),
    scratch_types=[
        pltpu.VMEM((CHUNK, D), jnp.float32),
        pltpu.VMEM((CHUNK, D), jnp.float32),
        pltpu.VMEM((16,), jnp.int32),
        pltpu.VMEM((B, D), jnp.float32),
        pltpu.SemaphoreType.DMA,
        pltpu.SemaphoreType.DMA,
    ],
)


def _tc_mlp_body(part_ref, len_ref, W1_ref, b1_ref, g1_ref, be1_ref,
                 W2_ref, b2_ref, g2_ref, be2_ref, out_ref):
    sums = jnp.sum(part_ref[:], axis=0)
    lenf = len_ref[:].astype(jnp.float32)
    pooled = sums / lenf[:, None]

    h = lax.dot_general(pooled, W1_ref[:], (((1,), (1,)), ((), ())),
                        preferred_element_type=jnp.float32) + b1_ref[:][None, :]
    mu = jnp.mean(h, axis=0)
    var = jnp.mean((h - mu) ** 2, axis=0)
    h = (h - mu) / jnp.sqrt(var + 1e-5) * g1_ref[:][None, :] + be1_ref[:][None, :]
    h = jnp.maximum(h, 0.0)

    h2 = lax.dot_general(h, W2_ref[:], (((1,), (1,)), ((), ())),
                         preferred_element_type=jnp.float32) + b2_ref[:][None, :]
    mu2 = jnp.mean(h2, axis=0)
    var2 = jnp.mean((h2 - mu2) ** 2, axis=0)
    out_ref[:] = ((h2 - mu2) / jnp.sqrt(var2 + 1e-5) * g2_ref[:][None, :]
                  + be2_ref[:][None, :])


def _tc_mlp(part, length, W1, b1, g1, be1, W2, b2, g2, be2):
    return pl.pallas_call(
        _tc_mlp_body,
        out_shape=jax.ShapeDtypeStruct((B, FC1), jnp.float32),
    )(part, length, W1, b1, g1, be1, W2, b2, g2, be2)


def kernel(x, length, W1, b1, g1, be1, W2, b2, g2, be2):
    part = _sc_segment_sums(x, length)
    return _tc_mlp(part, length, W1, b1, g1, be1, W2, b2, g2, be2)


# speedup vs baseline: 1.0059x; 1.0059x over previous
import jax
import jax.numpy as jnp
from jax import lax
from jax.experimental import pallas as pl
from jax.experimental.pallas import tpu as pltpu
from jax.experimental.pallas import tpu_sc as plsc

N, D, B = 32768, 32, 16
FC0, FC1 = 64, 128
NC, NS = 2, 16
NW = NC * NS
CHUNK = 256
NBUF = 2
NJ = N // (CHUNK * NW)
UNROLL = 4


def _sc_segment_sums_body(x_hbm, len_hbm, part_hbm,
                          b0, b1, len_v, part_v, s0, s1):
    c = lax.axis_index("c")
    s = lax.axis_index("s")
    wid = s * NC + c
    bufs = [b0, b1]
    sems = [s0, s1]

    zf = jnp.zeros((16,), jnp.float32)
    for b in range(B):
        part_v[b, 0:16] = zf
        part_v[b, 16:32] = zf

    pltpu.sync_copy(len_hbm, len_v)
    lv = len_v[:]
    cbs = []
    run = jnp.int32(0)
    for b in range(B):
        run = run + lv[b]
        cbs.append(run)
    total = cbs[B - 1]

    cbases = [(j * NW + wid) * CHUNK for j in range(NJ)]

    def issue(j):
        @pl.when(cbases[j] < total)
        def _():
            pltpu.async_copy(x_hbm.at[pl.ds(cbases[j], CHUNK)],
                             bufs[j % NBUF], sems[j % NBUF])

    for j in range(min(NBUF, NJ)):
        issue(j)

    for j in range(NJ):
        x_v = bufs[j % NBUF]

        @pl.when(cbases[j] < total)
        def _(j=j, x_v=x_v):
            pltpu.make_async_copy(x_hbm.at[pl.ds(cbases[j], CHUNK)],
                                  x_v, sems[j % NBUF]).wait()
            hbase = cbases[j]
            hi_h = hbase + CHUNK
            for b in range(B):
                lo_g = cbs[b - 1] if b else jnp.int32(0)
                lo = jnp.clip(lo_g, hbase, hi_h) - hbase
                hi = jnp.clip(cbs[b], hbase, hi_h) - hbase
                n = hi - lo

                @pl.when(n > 0)
                def _(b=b, lo=lo, hi=hi, n=n, x_v=x_v):
                    def u_body(t, accs, lo=lo):
                        r = lo + t * UNROLL
                        out = []
                        for u in range(UNROLL):
                            a0 = accs[2 * u] + x_v[r + u, 0:16]
                            a1 = accs[2 * u + 1] + x_v[r + u, 16:32]
                            out.extend((a0, a1))
                        return tuple(out)

                    n_main = n // UNROLL
                    accs = lax.fori_loop(0, n_main, u_body, (zf,) * (2 * UNROLL))

                    def r_body(r, accs2):
                        return (accs2[0] + x_v[r, 0:16],
                                accs2[1] + x_v[r, 16:32])

                    tail = lax.fori_loop(lo + n_main * UNROLL, hi, r_body,
                                         (zf, zf))

                    acc_lo = (accs[0] + accs[2]) + (accs[4] + accs[6]) + tail[0]
                    acc_hi = (accs[1] + accs[3]) + (accs[5] + accs[7]) + tail[1]
                    part_v[b, 0:16] = part_v[b, 0:16] + acc_lo
                    part_v[b, 16:32] = part_v[b, 16:32] + acc_hi

        if j + NBUF < NJ:
            issue(j + NBUF)

    pltpu.sync_copy(part_v, part_hbm.at[wid])


_sc_segment_sums = pl.kernel(
    _sc_segment_sums_body,
    out_type=jax.ShapeDtypeStruct((NW, B, D), jnp.float32),
    mesh=plsc.VectorSubcoreMesh(core_axis_name="c", subcore_axis_name="s"),
    scratch_types=[
        pltpu.VMEM((CHUNK, D), jnp.float32),
        pltpu.VMEM((CHUNK, D), jnp.float32),
        pltpu.VMEM((16,), jnp.int32),
        pltpu.VMEM((B, D), jnp.float32),
        pltpu.SemaphoreType.DMA,
        pltpu.SemaphoreType.DMA,
    ],
)


def _tc_mlp_body(part_ref, len_ref, W1_ref, b1_ref, g1_ref, be1_ref,
                 W2_ref, b2_ref, g2_ref, be2_ref, out_ref):
    sums = jnp.sum(part_ref[:], axis=0)
    lenf = len_ref[:].astype(jnp.float32)
    pooled = sums / lenf[:, None]

    h = lax.dot_general(pooled, W1_ref[:], (((1,), (1,)), ((), ())),
                        preferred_element_type=jnp.float32) + b1_ref[:][None, :]
    mu = jnp.mean(h, axis=0)
    var = jnp.mean((h - mu) ** 2, axis=0)
    h = (h - mu) / jnp.sqrt(var + 1e-5) * g1_ref[:][None, :] + be1_ref[:][None, :]
    h = jnp.maximum(h, 0.0)

    h2 = lax.dot_general(h, W2_ref[:], (((1,), (1,)), ((), ())),
                         preferred_element_type=jnp.float32) + b2_ref[:][None, :]
    mu2 = jnp.mean(h2, axis=0)
    var2 = jnp.mean((h2 - mu2) ** 2, axis=0)
    out_ref[:] = ((h2 - mu2) / jnp.sqrt(var2 + 1e-5) * g2_ref[:][None, :]
                  + be2_ref[:][None, :])


def _tc_mlp(part, length, W1, b1, g1, be1, W2, b2, g2, be2):
    return pl.pallas_call(
        _tc_mlp_body,
        out_shape=jax.ShapeDtypeStruct((B, FC1), jnp.float32),
    )(part, length, W1, b1, g1, be1, W2, b2, g2, be2)


def kernel(x, length, W1, b1, g1, be1, W2, b2, g2, be2):
    part = _sc_segment_sums(x, length)
    return _tc_mlp(part, length, W1, b1, g1, be1, W2, b2, g2, be2)
